# baseline (speedup 1.0000x reference)
import functools

import jax
import jax.numpy as jnp
from jax import lax
from jax.experimental import pallas as pl
from jax.experimental.pallas import tpu as pltpu
from jax.experimental.pallas import tpu_sc as plsc

_N = 10000
_E = 320000
_D = 128
_OUT = 64

_NPAD = 10240
_DUMMY = _NPAD - 1
_K = 64
_NSC = 2
_NTILE = 16
_NW = _NSC * _NTILE
_CHUNKS = 160
_EPT = _K * _CHUNKS
_EPAD = _EPT * _NW
_RPT = _NPAD // _NTILE
_BN = 512

_sc_mesh = plsc.VectorSubcoreMesh(core_axis_name="c", subcore_axis_name="s")


@functools.partial(
    pl.kernel,
    mesh=_sc_mesh,
    out_type=jax.ShapeDtypeStruct((_NSC, _NPAD), jnp.float32),
    scratch_types=[
        pltpu.VMEM((_CHUNKS, _K), jnp.int32),
        pltpu.VMEM((_K,), jnp.float32),
        pltpu.VMEM((_RPT,), jnp.float32),
        pltpu.VMEM_SHARED((_NPAD,), jnp.float32),
    ],
)
def _deg_kernel(dst_hbm, out_hbm, didx, ones_v, zbuf, acc):
    cid = lax.axis_index("c")
    sid = lax.axis_index("s")
    wid = cid * _NTILE + sid

    def fill_ones(i, _):
        ones_v[pl.ds(i * 16, 16)] = jnp.ones((16,), jnp.float32)
        return 0

    lax.fori_loop(0, _K // 16, fill_ones, 0)

    def fill_zero(i, _):
        zbuf[pl.ds(i * 16, 16)] = jnp.zeros((16,), jnp.float32)
        return 0

    lax.fori_loop(0, _RPT // 16, fill_zero, 0)
    pltpu.sync_copy(zbuf, acc.at[pl.ds(sid * _RPT, _RPT)])
    pltpu.sync_copy(dst_hbm.at[pl.ds(wid * _CHUNKS, _CHUNKS)], didx)
    plsc.subcore_barrier()

    def chunk(j, _):
        pltpu.sync_copy(ones_v, acc.at[didx.at[j]], add=True)
        return 0

    lax.fori_loop(0, _CHUNKS, chunk, 0)
    plsc.subcore_barrier()
    pltpu.sync_copy(acc.at[pl.ds(sid * _RPT, _RPT)],
                    out_hbm.at[cid, pl.ds(sid * _RPT, _RPT)])


_SUP = 16
_NSUP = _CHUNKS // _SUP

_C0 = 192
_C1 = 128


@functools.partial(
    pl.kernel,
    mesh=_sc_mesh,
    out_type=jax.ShapeDtypeStruct((_NSC, _NPAD, _D), jnp.float32),
    scratch_types=[
        pltpu.VMEM((2, _SUP, _K), jnp.int32),
        pltpu.VMEM((2, _SUP, _K), jnp.int32),
        pltpu.VMEM((4, _K, _D), jnp.float32),
        pltpu.VMEM((32, _D), jnp.float32),
        pltpu.VMEM_SHARED((_NPAD, _D), jnp.float32),
        pltpu.SemaphoreType.DMA,
        pltpu.SemaphoreType.DMA,
        pltpu.SemaphoreType.DMA,
        pltpu.SemaphoreType.DMA,
        pltpu.SemaphoreType.DMA,
        pltpu.SemaphoreType.DMA,
        pltpu.SemaphoreType.DMA,
        pltpu.SemaphoreType.DMA,
        pltpu.SemaphoreType.DMA,
    ],
)
def _scatter_kernel(g_hbm, src_hbm, dst_hbm, out_hbm,
                    sidx, didx, rows, zbuf, acc,
                    gsem0, gsem1, gsem2, gsem3,
                    ssem0, ssem1, ssem2, ssem3, isem):
    cid = lax.axis_index("c")
    sid = lax.axis_index("s")
    gsems = (gsem0, gsem1, gsem2, gsem3)
    ssems = (ssem0, ssem1, ssem2, ssem3)
    cbase = lax.select(cid == 0, sid * _C0, _NTILE * _C0 + sid * _C1)
    nsup = lax.select(cid == 0, _C0 // _SUP, _C1 // _SUP)

    def idx_start(s, b):
        base = cbase + s * _SUP
        pltpu.async_copy(src_hbm.at[pl.ds(base, _SUP)], sidx.at[b], isem)
        pltpu.async_copy(dst_hbm.at[pl.ds(base, _SUP)], didx.at[b], isem)

    def idx_wait(s, b):
        base = cbase + s * _SUP
        pltpu.make_async_copy(src_hbm.at[pl.ds(base, _SUP)], sidx.at[b], isem).wait()
        pltpu.make_async_copy(dst_hbm.at[pl.ds(base, _SUP)], didx.at[b], isem).wait()

    def fill_zero(i, _):
        for j in range(_D // 16):
            zbuf[i, pl.ds(j * 16, 16)] = jnp.zeros((16,), jnp.float32)
        return 0

    lax.fori_loop(0, 32, fill_zero, 0)
    idx_start(0, 0)
    _nz = _RPT // 32
    for t in range(_nz):
        pltpu.async_copy(zbuf, acc.at[pl.ds(sid * _RPT + t * 32, 32)],
                         ssems[t % 4])
    for t in range(_nz):
        pltpu.make_async_copy(zbuf, acc.at[pl.ds(sid * _RPT + t * 32, 32)],
                              ssems[t % 4]).wait()
    plsc.subcore_barrier()
    idx_wait(0, 0)

    def gather_start(b, c, rb):
        pltpu.async_copy(g_hbm.at[sidx.at[b, c]], rows.at[rb], gsems[rb])

    def gather_wait(b, c, rb):
        pltpu.make_async_copy(g_hbm.at[sidx.at[b, c]], rows.at[rb], gsems[rb]).wait()

    def do_super(s, b):
        @pl.when(s + 1 < nsup)
        def _():
            idx_start(s + 1, 1 - b)

        pend = {}
        for c in range(3):
            gather_start(b, c, c)
        for c in range(_SUP):
            rb = c % 4
            if c + 3 < _SUP:
                if c - 1 >= 0:
                    pend.pop(c - 1).wait()
                gather_start(b, c + 3, (c + 3) % 4)
            gather_wait(b, c, rb)
            pend[c] = pltpu.async_copy(rows.at[rb], acc.at[didx.at[b, c]],
                                       ssems[rb], add=True)
        for c in range(_SUP - 4, _SUP):
            pend.pop(c).wait()

        @pl.when(s + 1 < nsup)
        def _():
            idx_wait(s + 1, 1 - b)

    def outer(t, _):
        do_super(2 * t, 0)
        do_super(2 * t + 1, 1)
        return 0

    lax.fori_loop(0, nsup // 2, outer, 0)
    plsc.subcore_barrier()
    pltpu.sync_copy(acc.at[pl.ds(sid * _RPT, _RPT)],
                    out_hbm.at[cid, pl.ds(sid * _RPT, _RPT)])


def _dinv_body(p_ref, o_ref):
    o_ref[...] = lax.rsqrt(p_ref[0] + p_ref[1] + 1.0)


_dinv_call = pl.pallas_call(
    _dinv_body,
    out_shape=jax.ShapeDtypeStruct((_NPAD // 128, 128), jnp.float32),
)


def _lin1_body(dinv_ref, x_ref, w_ref, g_ref):
    h = jnp.dot(x_ref[...], w_ref[...], preferred_element_type=jnp.float32)
    g_ref[...] = h * dinv_ref[...]


_lin1_call = pl.pallas_call(
    _lin1_body,
    grid=(_NPAD // _BN,),
    in_specs=[
        pl.BlockSpec((_BN, 1), lambda i: (i, 0)),
        pl.BlockSpec((_BN, _D), lambda i: (i, 0)),
        pl.BlockSpec((_D, _D), lambda i: (0, 0)),
    ],
    out_specs=pl.BlockSpec((_BN, _D), lambda i: (i, 0)),
    out_shape=jax.ShapeDtypeStruct((_NPAD, _D), jnp.float32),
)


def _lin2_body(dinv_ref, p_ref, g1_ref, b_ref, w_ref, g2_ref):
    s = p_ref[0] + p_ref[1] + g1_ref[...]
    x2 = jnp.maximum(s * dinv_ref[...] + b_ref[...], 0.0)
    g2_ref[...] = (jnp.dot(x2, w_ref[...], preferred_element_type=jnp.float32)
                   * dinv_ref[...])


_lin2_call = pl.pallas_call(
    _lin2_body,
    grid=(_NPAD // _BN,),
    in_specs=[
        pl.BlockSpec((_BN, 1), lambda i: (i, 0)),
        pl.BlockSpec((_NSC, _BN, _D), lambda i: (0, i, 0)),
        pl.BlockSpec((_BN, _D), lambda i: (i, 0)),
        pl.BlockSpec((1, _D), lambda i: (0, 0)),
        pl.BlockSpec((_D, _D), lambda i: (0, 0)),
    ],
    out_specs=pl.BlockSpec((_BN, _D), lambda i: (i, 0)),
    out_shape=jax.ShapeDtypeStruct((_NPAD, _D), jnp.float32),
)


def _head_body(dinv_ref, p_ref, g2_ref, b2_ref, wp1_ref, bp1_ref,
               wp2_ref, bp2_ref, emb_ref, out_ref):
    s = p_ref[0] + p_ref[1] + g2_ref[...]
    emb = s * dinv_ref[...] + b2_ref[...]
    emb_ref[...] = emb
    x3 = jnp.maximum(emb, 0.0)
    h3 = jnp.dot(x3, wp1_ref[...], preferred_element_type=jnp.float32) + bp1_ref[...]
    h4 = jnp.dot(h3, wp2_ref[...], preferred_element_type=jnp.float32) + bp2_ref[...]
    m = jnp.max(h4, axis=1, keepdims=True)
    e = h4 - m
    lse = jnp.log(jnp.sum(jnp.exp(e), axis=1, keepdims=True))
    out_ref[...] = e - lse


_head_call = pl.pallas_call(
    _head_body,
    grid=(_NPAD // _BN,),
    in_specs=[
        pl.BlockSpec((_BN, 1), lambda i: (i, 0)),
        pl.BlockSpec((_NSC, _BN, _D), lambda i: (0, i, 0)),
        pl.BlockSpec((_BN, _D), lambda i: (i, 0)),
        pl.BlockSpec((1, _D), lambda i: (0, 0)),
        pl.BlockSpec((_D, _D), lambda i: (0, 0)),
        pl.BlockSpec((1, _D), lambda i: (0, 0)),
        pl.BlockSpec((_D, _OUT), lambda i: (0, 0)),
        pl.BlockSpec((1, _OUT), lambda i: (0, 0)),
    ],
    out_specs=[
        pl.BlockSpec((_BN, _D), lambda i: (i, 0)),
        pl.BlockSpec((_BN, _OUT), lambda i: (i, 0)),
    ],
    out_shape=[
        jax.ShapeDtypeStruct((_NPAD, _D), jnp.float32),
        jax.ShapeDtypeStruct((_NPAD, _OUT), jnp.float32),
    ],
)


def kernel(x, edge_index, batch, W1, b1, W2, b2, Wp1, bp1, Wp2, bp2):
    src = edge_index[0]
    dst = edge_index[1]
    pad = _EPAD - _E
    src_p = jnp.concatenate(
        [src, jnp.zeros((pad,), jnp.int32)]).reshape(_EPAD // _K, _K)
    dst_p = jnp.concatenate(
        [dst, jnp.full((pad,), _DUMMY, jnp.int32)]).reshape(_EPAD // _K, _K)
    x_p = jnp.pad(x, ((0, _NPAD - _N), (0, 0)))

    degp = _deg_kernel(dst_p)
    dinv = _dinv_call(degp.reshape(_NSC, _NPAD // 128, 128)).reshape(_NPAD, 1)
    g1 = _lin1_call(dinv, x_p, W1)
    p1 = _scatter_kernel(g1, src_p, dst_p)
    g2 = _lin2_call(dinv, p1, g1, b1.reshape(1, _D), W2)
    p2 = _scatter_kernel(g2, src_p, dst_p)
    emb, out = _head_call(dinv, p2, g2, b2.reshape(1, _D),
                          Wp1, bp1.reshape(1, _D), Wp2, bp2.reshape(1, _OUT))
    return (emb[:_N], out[:_N])


# speedup vs baseline: 1.2828x; 1.2828x over previous
import functools

import jax
import jax.numpy as jnp
from jax import lax
from jax.experimental import pallas as pl
from jax.experimental.pallas import tpu as pltpu
from jax.experimental.pallas import tpu_sc as plsc

_N = 10000
_E = 320000
_D = 128
_OUT = 64

_NPAD = 10240
_DUMMY = _NPAD - 1
_K = 64
_NSC = 2
_NTILE = 16
_NW = _NSC * _NTILE
_CHUNKS = 160
_EPT = _K * _CHUNKS
_EPAD = _EPT * _NW
_RPT = _NPAD // _NTILE
_BN = 512

_sc_mesh = plsc.VectorSubcoreMesh(core_axis_name="c", subcore_axis_name="s")


@functools.partial(
    pl.kernel,
    mesh=_sc_mesh,
    out_type=jax.ShapeDtypeStruct((_NSC, _NPAD), jnp.float32),
    scratch_types=[
        pltpu.VMEM((_CHUNKS, _K), jnp.int32),
        pltpu.VMEM((_K,), jnp.float32),
        pltpu.VMEM((_RPT,), jnp.float32),
        pltpu.VMEM_SHARED((_NPAD,), jnp.float32),
    ],
)
def _deg_kernel(dst_hbm, out_hbm, didx, ones_v, zbuf, acc):
    cid = lax.axis_index("c")
    sid = lax.axis_index("s")
    wid = cid * _NTILE + sid

    def fill_ones(i, _):
        ones_v[pl.ds(i * 16, 16)] = jnp.ones((16,), jnp.float32)
        return 0

    lax.fori_loop(0, _K // 16, fill_ones, 0)

    def fill_zero(i, _):
        zbuf[pl.ds(i * 16, 16)] = jnp.zeros((16,), jnp.float32)
        return 0

    lax.fori_loop(0, _RPT // 16, fill_zero, 0)
    pltpu.sync_copy(zbuf, acc.at[pl.ds(sid * _RPT, _RPT)])
    pltpu.sync_copy(dst_hbm.at[pl.ds(wid * _CHUNKS, _CHUNKS)], didx)
    plsc.subcore_barrier()

    def chunk(j, _):
        pltpu.sync_copy(ones_v, acc.at[didx.at[j]], add=True)
        return 0

    lax.fori_loop(0, _CHUNKS, chunk, 0)
    plsc.subcore_barrier()
    pltpu.sync_copy(acc.at[pl.ds(sid * _RPT, _RPT)],
                    out_hbm.at[cid, pl.ds(sid * _RPT, _RPT)])


_SUP = 16
_NSUP = _CHUNKS // _SUP

_C0 = 240
_C1 = 80


@functools.partial(
    pl.kernel,
    mesh=_sc_mesh,
    out_type=jax.ShapeDtypeStruct((_NSC, _NPAD, _D), jnp.float32),
    scratch_types=[
        pltpu.VMEM((2, _SUP, _K), jnp.int32),
        pltpu.VMEM((2, _SUP, _K), jnp.int32),
        pltpu.VMEM((4, _K, _D), jnp.float32),
        pltpu.VMEM((32, _D), jnp.float32),
        pltpu.VMEM_SHARED((_NPAD, _D), jnp.float32),
        pltpu.SemaphoreType.DMA,
        pltpu.SemaphoreType.DMA,
        pltpu.SemaphoreType.DMA,
        pltpu.SemaphoreType.DMA,
        pltpu.SemaphoreType.DMA,
        pltpu.SemaphoreType.DMA,
        pltpu.SemaphoreType.DMA,
        pltpu.SemaphoreType.DMA,
        pltpu.SemaphoreType.DMA,
    ],
)
def _scatter_kernel(g_hbm, src_hbm, dst_hbm, out_hbm,
                    sidx, didx, rows, zbuf, acc,
                    gsem0, gsem1, gsem2, gsem3,
                    ssem0, ssem1, ssem2, ssem3, isem):
    cid = lax.axis_index("c")
    sid = lax.axis_index("s")
    gsems = (gsem0, gsem1, gsem2, gsem3)
    ssems = (ssem0, ssem1, ssem2, ssem3)
    cbase = lax.select(cid == 0, sid * _C0, _NTILE * _C0 + sid * _C1)
    nsup = lax.select(cid == 0, _C0 // _SUP, _C1 // _SUP)

    def idx_start(s, b):
        base = cbase + s * _SUP
        pltpu.async_copy(src_hbm.at[pl.ds(base, _SUP)], sidx.at[b], isem)
        pltpu.async_copy(dst_hbm.at[pl.ds(base, _SUP)], didx.at[b], isem)

    def idx_wait(s, b):
        base = cbase + s * _SUP
        pltpu.make_async_copy(src_hbm.at[pl.ds(base, _SUP)], sidx.at[b], isem).wait()
        pltpu.make_async_copy(dst_hbm.at[pl.ds(base, _SUP)], didx.at[b], isem).wait()

    def fill_zero(i, _):
        for j in range(_D // 16):
            zbuf[i, pl.ds(j * 16, 16)] = jnp.zeros((16,), jnp.float32)
        return 0

    lax.fori_loop(0, 32, fill_zero, 0)
    idx_start(0, 0)
    _nz = _RPT // 32
    for t in range(_nz):
        pltpu.async_copy(zbuf, acc.at[pl.ds(sid * _RPT + t * 32, 32)],
                         ssems[t % 4])
    for t in range(_nz):
        pltpu.make_async_copy(zbuf, acc.at[pl.ds(sid * _RPT + t * 32, 32)],
                              ssems[t % 4]).wait()
    plsc.subcore_barrier()
    idx_wait(0, 0)

    def gather_start(b, c, rb):
        pltpu.async_copy(g_hbm.at[sidx.at[b, c]], rows.at[rb], gsems[rb])

    def gather_wait(b, c, rb):
        pltpu.make_async_copy(g_hbm.at[sidx.at[b, c]], rows.at[rb], gsems[rb]).wait()

    def do_super(s, b):
        @pl.when(s + 1 < nsup)
        def _():
            idx_start(s + 1, 1 - b)

        pend = {}
        for c in range(3):
            gather_start(b, c, c)
        for c in range(_SUP):
            rb = c % 4
            if c + 3 < _SUP:
                if c - 1 >= 0:
                    pend.pop(c - 1).wait()
                gather_start(b, c + 3, (c + 3) % 4)
            gather_wait(b, c, rb)
            pend[c] = pltpu.async_copy(rows.at[rb], acc.at[didx.at[b, c]],
                                       ssems[rb], add=True)
        for c in range(_SUP - 4, _SUP):
            pend.pop(c).wait()

        @pl.when(s + 1 < nsup)
        def _():
            idx_wait(s + 1, 1 - b)

    def outer(t, _):
        do_super(2 * t, 0)
        do_super(2 * t + 1, 1)
        return 0

    lax.fori_loop(0, nsup // 2, outer, 0)
    plsc.subcore_barrier()
    pltpu.sync_copy(acc.at[pl.ds(sid * _RPT, _RPT)],
                    out_hbm.at[cid, pl.ds(sid * _RPT, _RPT)])


def _dinv_body(p_ref, o_ref):
    o_ref[...] = lax.rsqrt(p_ref[0] + p_ref[1] + 1.0)


_dinv_call = pl.pallas_call(
    _dinv_body,
    out_shape=jax.ShapeDtypeStruct((_NPAD // 128, 128), jnp.float32),
)


def _lin1_body(dinv_ref, x_ref, w_ref, g_ref):
    h = jnp.dot(x_ref[...], w_ref[...], preferred_element_type=jnp.float32)
    g_ref[...] = h * dinv_ref[...]


_lin1_call = pl.pallas_call(
    _lin1_body,
    grid=(_NPAD // _BN,),
    in_specs=[
        pl.BlockSpec((_BN, 1), lambda i: (i, 0)),
        pl.BlockSpec((_BN, _D), lambda i: (i, 0)),
        pl.BlockSpec((_D, _D), lambda i: (0, 0)),
    ],
    out_specs=pl.BlockSpec((_BN, _D), lambda i: (i, 0)),
    out_shape=jax.ShapeDtypeStruct((_NPAD, _D), jnp.float32),
)


def _lin2_body(dinv_ref, p_ref, g1_ref, b_ref, w_ref, g2_ref):
    s = p_ref[0] + p_ref[1] + g1_ref[...]
    x2 = jnp.maximum(s * dinv_ref[...] + b_ref[...], 0.0)
    g2_ref[...] = (jnp.dot(x2, w_ref[...], preferred_element_type=jnp.float32)
                   * dinv_ref[...])


_lin2_call = pl.pallas_call(
    _lin2_body,
    grid=(_NPAD // _BN,),
    in_specs=[
        pl.BlockSpec((_BN, 1), lambda i: (i, 0)),
        pl.BlockSpec((_NSC, _BN, _D), lambda i: (0, i, 0)),
        pl.BlockSpec((_BN, _D), lambda i: (i, 0)),
        pl.BlockSpec((1, _D), lambda i: (0, 0)),
        pl.BlockSpec((_D, _D), lambda i: (0, 0)),
    ],
    out_specs=pl.BlockSpec((_BN, _D), lambda i: (i, 0)),
    out_shape=jax.ShapeDtypeStruct((_NPAD, _D), jnp.float32),
)


def _head_body(dinv_ref, p_ref, g2_ref, b2_ref, wp1_ref, bp1_ref,
               wp2_ref, bp2_ref, emb_ref, out_ref):
    s = p_ref[0] + p_ref[1] + g2_ref[...]
    emb = s * dinv_ref[...] + b2_ref[...]
    emb_ref[...] = emb
    x3 = jnp.maximum(emb, 0.0)
    h3 = jnp.dot(x3, wp1_ref[...], preferred_element_type=jnp.float32) + bp1_ref[...]
    h4 = jnp.dot(h3, wp2_ref[...], preferred_element_type=jnp.float32) + bp2_ref[...]
    m = jnp.max(h4, axis=1, keepdims=True)
    e = h4 - m
    lse = jnp.log(jnp.sum(jnp.exp(e), axis=1, keepdims=True))
    out_ref[...] = e - lse


_head_call = pl.pallas_call(
    _head_body,
    grid=(_NPAD // _BN,),
    in_specs=[
        pl.BlockSpec((_BN, 1), lambda i: (i, 0)),
        pl.BlockSpec((_NSC, _BN, _D), lambda i: (0, i, 0)),
        pl.BlockSpec((_BN, _D), lambda i: (i, 0)),
        pl.BlockSpec((1, _D), lambda i: (0, 0)),
        pl.BlockSpec((_D, _D), lambda i: (0, 0)),
        pl.BlockSpec((1, _D), lambda i: (0, 0)),
        pl.BlockSpec((_D, _OUT), lambda i: (0, 0)),
        pl.BlockSpec((1, _OUT), lambda i: (0, 0)),
    ],
    out_specs=[
        pl.BlockSpec((_BN, _D), lambda i: (i, 0)),
        pl.BlockSpec((_BN, _OUT), lambda i: (i, 0)),
    ],
    out_shape=[
        jax.ShapeDtypeStruct((_NPAD, _D), jnp.float32),
        jax.ShapeDtypeStruct((_NPAD, _OUT), jnp.float32),
    ],
)


def kernel(x, edge_index, batch, W1, b1, W2, b2, Wp1, bp1, Wp2, bp2):
    src = edge_index[0]
    dst = edge_index[1]
    pad = _EPAD - _E
    src_p = jnp.concatenate(
        [src, jnp.zeros((pad,), jnp.int32)]).reshape(_EPAD // _K, _K)
    dst_p = jnp.concatenate(
        [dst, jnp.full((pad,), _DUMMY, jnp.int32)]).reshape(_EPAD // _K, _K)
    x_p = jnp.pad(x, ((0, _NPAD - _N), (0, 0)))

    degp = _deg_kernel(dst_p)
    dinv = _dinv_call(degp.reshape(_NSC, _NPAD // 128, 128)).reshape(_NPAD, 1)
    g1 = _lin1_call(dinv, x_p, W1)
    p1 = _scatter_kernel(g1, src_p, dst_p)
    g2 = _lin2_call(dinv, p1, g1, b1.reshape(1, _D), W2)
    p2 = _scatter_kernel(g2, src_p, dst_p)
    emb, out = _head_call(dinv, p2, g2, b2.reshape(1, _D),
                          Wp1, bp1.reshape(1, _D), Wp2, bp2.reshape(1, _OUT))
    return (emb[:_N], out[:_N])
